# baseline (speedup 1.0000x reference)
import functools
import math

import jax
import jax.numpy as jnp
from jax import lax
from jax.experimental import pallas as pl
from jax.experimental.pallas import tpu as pltpu
from jax.experimental.pallas import tpu_sc as plsc

D_MODEL = 64
SCALE = math.sqrt(D_MODEL)

NC = 2
NS = 16
NW = NC * NS
G = 128
STEPS = 200

_mesh = plsc.VectorSubcoreMesh(core_axis_name="c", subcore_axis_name="s")


@functools.partial(
    pl.kernel,
    mesh=_mesh,
    compiler_params=pltpu.CompilerParams(use_tc_tiling_on_sc=False),
    out_type=jax.ShapeDtypeStruct((NW * STEPS, G, D_MODEL), jnp.float32),
    scratch_types=[
        pltpu.VMEM((STEPS, G), jnp.int32),
        pltpu.VMEM((G, D_MODEL), jnp.float32),
        pltpu.VMEM((G, D_MODEL), jnp.float32),
        pltpu.SemaphoreType.DMA,
    ],
)
def _emb_lookup(table_hbm, idx_hbm, out_hbm, idx_v, inb, outb, sem):
    wid = lax.axis_index("s") * NC + lax.axis_index("c")
    pltpu.sync_copy(idx_hbm.at[wid], idx_v)

    def step(i, carry):
        pltpu.async_copy(table_hbm.at[idx_v.at[i]], inb, sem).wait()

        def row(r, c):
            for j in range(D_MODEL // 16):
                sl = pl.ds(j * 16, 16)
                outb[r, sl] = inb[r, sl] * SCALE
            return c

        lax.fori_loop(0, G, row, 0)
        pltpu.sync_copy(outb, out_hbm.at[wid * STEPS + i])
        return carry

    lax.fori_loop(0, STEPS, step, 0)


def kernel(x, table):
    n_tok = x.shape[0] * x.shape[1]
    xi = x.astype(jnp.int32).reshape(NW, STEPS, G)
    out = _emb_lookup(table, xi)
    return out.reshape(x.shape[0], x.shape[1], D_MODEL)


# speedup vs baseline: 1.0733x; 1.0733x over previous
import functools
import math

import jax
import jax.numpy as jnp
from jax import lax
from jax.experimental import pallas as pl
from jax.experimental.pallas import tpu as pltpu
from jax.experimental.pallas import tpu_sc as plsc

D_MODEL = 64
SCALE = math.sqrt(D_MODEL)

NC = 2
NS = 16
NW = NC * NS
G = 128
STEPS = 200
NBUF = 4

_mesh = plsc.VectorSubcoreMesh(core_axis_name="c", subcore_axis_name="s")


@functools.partial(
    pl.kernel,
    mesh=_mesh,
    compiler_params=pltpu.CompilerParams(use_tc_tiling_on_sc=False),
    out_type=jax.ShapeDtypeStruct((NW * STEPS, G, D_MODEL), jnp.float32),
    scratch_types=[
        pltpu.VMEM((STEPS, G), jnp.int32),
        pltpu.VMEM((NBUF, G, D_MODEL), jnp.float32),
        pltpu.VMEM((NBUF, G, D_MODEL), jnp.float32),
        pltpu.SemaphoreType.DMA((NBUF,)),
        pltpu.SemaphoreType.DMA((NBUF,)),
    ],
)
def _emb_lookup(table_hbm, idx_hbm, out_hbm, idx_v, inb, outb, gsem, ssem):
    wid = lax.axis_index("s") * NC + lax.axis_index("c")
    obase = wid * STEPS
    pltpu.sync_copy(idx_hbm.at[wid], idx_v)

    for b in range(NBUF):
        pltpu.make_async_copy(
            table_hbm.at[idx_v.at[b]], inb.at[b], gsem.at[b]
        ).start()

    def group(g, carry):
        for b in range(NBUF):
            i = g * NBUF + b
            pltpu.make_async_copy(
                table_hbm.at[idx_v.at[i]], inb.at[b], gsem.at[b]
            ).wait()

            @pl.when(g > 0)
            def _wait_store():
                pltpu.make_async_copy(
                    outb.at[b], out_hbm.at[obase], ssem.at[b]
                ).wait()

            def row(r, c):
                for j in range(D_MODEL // 16):
                    sl = pl.ds(j * 16, 16)
                    outb[b, r, sl] = inb[b, r, sl] * SCALE
                return c

            lax.fori_loop(0, G, row, 0, unroll=4)

            pltpu.make_async_copy(
                outb.at[b], out_hbm.at[obase + i], ssem.at[b]
            ).start()

            @pl.when(g < STEPS // NBUF - 1)
            def _next_gather():
                pltpu.make_async_copy(
                    table_hbm.at[idx_v.at[i + NBUF]], inb.at[b], gsem.at[b]
                ).start()

        return carry

    lax.fori_loop(0, STEPS // NBUF, group, 0)

    for b in range(NBUF):
        pltpu.make_async_copy(
            outb.at[b], out_hbm.at[obase], ssem.at[b]
        ).wait()


def kernel(x, table):
    xi = x.astype(jnp.int32).reshape(NW, STEPS, G)
    out = _emb_lookup(table, xi)
    return out.reshape(x.shape[0], x.shape[1], D_MODEL)


# speedup vs baseline: 1.1717x; 1.0916x over previous
import functools
import math

import jax
import jax.numpy as jnp
from jax import lax
from jax.experimental import pallas as pl
from jax.experimental.pallas import tpu as pltpu
from jax.experimental.pallas import tpu_sc as plsc

D_MODEL = 64
DPAD = 128
SCALE = math.sqrt(D_MODEL)

NC = 2
NS = 16
NW = NC * NS
G = 128
STEPS = 200
NBUF = 2

_mesh = plsc.VectorSubcoreMesh(core_axis_name="c", subcore_axis_name="s")


@functools.partial(
    pl.kernel,
    mesh=_mesh,
    out_type=jax.ShapeDtypeStruct((4096, STEPS, D_MODEL), jnp.float32),
    scratch_types=[
        pltpu.VMEM((STEPS, G), jnp.int32),
        pltpu.VMEM((NBUF, G, DPAD), jnp.float32),
        pltpu.VMEM((NBUF, G, D_MODEL), jnp.float32),
        pltpu.SemaphoreType.DMA((NBUF,)),
        pltpu.SemaphoreType.DMA((NBUF,)),
    ],
)
def _emb_lookup(table_hbm, idx_hbm, out_hbm, idx_v, inb, outb, gsem, ssem):
    wid = lax.axis_index("s") * NC + lax.axis_index("c")
    bbase = wid * G
    pltpu.sync_copy(idx_hbm.at[:, pl.ds(bbase, G)], idx_v)

    for b in range(NBUF):
        pltpu.make_async_copy(
            table_hbm.at[idx_v.at[b]], inb.at[b], gsem.at[b]
        ).start()

    def group(g, carry):
        for b in range(NBUF):
            i = g * NBUF + b
            pltpu.make_async_copy(
                table_hbm.at[idx_v.at[i]], inb.at[b], gsem.at[b]
            ).wait()

            @pl.when(g > 0)
            def _wait_store():
                pltpu.make_async_copy(
                    outb.at[b], out_hbm.at[pl.ds(bbase, G), 0], ssem.at[b]
                ).wait()

            def row(r, c):
                for j in range(D_MODEL // 16):
                    sl = pl.ds(j * 16, 16)
                    outb[b, r, sl] = inb[b, r, sl] * SCALE
                return c

            lax.fori_loop(0, G, row, 0, unroll=4)

            pltpu.make_async_copy(
                outb.at[b], out_hbm.at[pl.ds(bbase, G), i], ssem.at[b]
            ).start()

            @pl.when(g < STEPS // NBUF - 1)
            def _next_gather():
                pltpu.make_async_copy(
                    table_hbm.at[idx_v.at[i + NBUF]], inb.at[b], gsem.at[b]
                ).start()

        return carry

    lax.fori_loop(0, STEPS // NBUF, group, 0)

    for b in range(NBUF):
        pltpu.make_async_copy(
            outb.at[b], out_hbm.at[pl.ds(bbase, G), 0], ssem.at[b]
        ).wait()


def kernel(x, table):
    tpad = jnp.pad(table, ((0, 0), (0, DPAD - D_MODEL)))
    xt = jnp.transpose(x)
    out = _emb_lookup(tpad, xt)
    return out
